# Initial kernel scaffold; baseline (speedup 1.0000x reference)
import jax
import jax.numpy as jnp
from jax.experimental import pallas as pl


def kernel(patient, treatment, confounders, primitive_corpus, W1, b1, W2, b2, Wt, bt, Wc, bc, Wa1, ba1, Wa2, ba2, Wo1, bo1, Wo2, bo2, Wo3, bo3, Wat1, bat1, Wat2, bat2):
    raise NotImplementedError("write your pallas kernel here")


# speedup vs baseline: 2.0732x; 2.0732x over previous
import functools

import jax
import jax.numpy as jnp
from jax import lax
from jax.experimental import pallas as pl
from jax.experimental.pallas import tpu as pltpu
from jax.experimental.pallas import tpu_sc as plsc

_B = 1024
_K = 100000
_D = 64
_TOPK = 8
_BK = 2048
_KPAD = 100352
_NB = _KPAD // _BK
_NEG = float("-inf")
_BIGI = 2**30


def _topk_body(patient_ref, w1_ref, b1_ref, w2_ref, b2_ref, corpus_ref,
               scores_ref, idx_ref, pe_ref, ts_ref, ti_ref):
    j = pl.program_id(0)

    @pl.when(j == 0)
    def _init():
        h = jnp.maximum(patient_ref[...] @ w1_ref[...] + b1_ref[...], 0.0)
        pe = h @ w2_ref[...] + b2_ref[...]
        n = jnp.sqrt(jnp.sum(pe * pe, axis=1, keepdims=True))
        pe_ref[...] = pe / jnp.maximum(n, 1e-12)
        ts_ref[...] = jnp.full((_B, 128), _NEG, jnp.float32)
        ti_ref[...] = jnp.full((_B, 128), _BIGI, jnp.int32)

    ce = corpus_ref[...]
    n = jnp.sqrt(jnp.sum(ce * ce, axis=1, keepdims=True))
    cen = ce / jnp.maximum(n, 1e-12)
    sim = lax.dot_general(pe_ref[...], cen, (((1,), (1,)), ((), ())),
                          preferred_element_type=jnp.float32)
    col = j * _BK + lax.broadcasted_iota(jnp.int32, (_B, _BK), 1)
    valid = col < _K
    sim = jnp.where(valid, sim, _NEG)
    col = jnp.where(valid, col, _BIGI)

    cand_s = jnp.concatenate([ts_ref[...], sim], axis=1)
    cand_i = jnp.concatenate([ti_ref[...], col], axis=1)

    out_s = []
    out_i = []
    for _ in range(_TOPK):
        m = jnp.max(cand_s, axis=1, keepdims=True)
        pick = jnp.min(jnp.where(cand_s == m, cand_i, _BIGI), axis=1,
                       keepdims=True)
        out_s.append(m)
        out_i.append(pick)
        cand_s = jnp.where(cand_i == pick, _NEG, cand_s)
    top_s = jnp.concatenate(out_s, axis=1)
    top_i = jnp.concatenate(out_i, axis=1)
    ts_ref[:, 0:_TOPK] = top_s
    ti_ref[:, 0:_TOPK] = top_i

    @pl.when(j == _NB - 1)
    def _emit():
        scores_ref[...] = top_s
        idx_ref[...] = top_i


def _tail_body(flat_ref, treat_ref, conf_ref,
               wt_ref, bt_ref, wc_ref, bc_ref,
               wa1_ref, ba1_ref, wa2_ref, ba2_ref,
               wo1_ref, bo1_ref, wo2_ref, bo2_ref, wo3_ref, bo3_ref,
               wat1_ref, bat1_ref, wat2_ref, bat2_ref,
               outcome_ref, attr_ref):
    f32 = jnp.float32
    mm = lambda a, b: lax.dot_general(a, b, (((1,), (0,)), ((), ())),
                                      preferred_element_type=f32)
    te = mm(treat_ref[...], wt_ref[...]) + bt_ref[...]
    cenc = mm(conf_ref[...], wc_ref[...]) + bc_ref[...]
    penc = mm(jnp.maximum(mm(flat_ref[...], wa1_ref[...]) + ba1_ref[...], 0.0),
              wa2_ref[...]) + ba2_ref[...]
    h = jnp.maximum(mm(te, wo1_ref[0:256, :]) + mm(cenc, wo1_ref[256:512, :])
                    + mm(penc, wo1_ref[512:768, :]) + bo1_ref[...], 0.0)
    h2 = jnp.maximum(mm(h, wo2_ref[...]) + bo2_ref[...], 0.0)
    outcome = mm(h2, wo3_ref[...]) + bo3_ref[...]
    outcome_ref[...] = outcome
    a = jnp.maximum(mm(penc, wat1_ref[0:256, :])
                    + mm(outcome, wat1_ref[256:264, :]) + bat1_ref[...], 0.0)
    logits = mm(a, wat2_ref[...]) + bat2_ref[...]
    mx = jnp.max(logits, axis=1, keepdims=True)
    e = jnp.exp(logits - mx)
    attr_ref[...] = e / jnp.sum(e, axis=1, keepdims=True)


def _sc_gather(table, idx):
    nrows = idx.shape[0]
    info = plsc.get_sparse_core_info()
    nw = info.num_cores * info.num_subcores
    per_w = nrows // nw
    mesh = plsc.VectorSubcoreMesh(core_axis_name="c", subcore_axis_name="s")

    @functools.partial(
        pl.kernel, mesh=mesh,
        out_type=jax.ShapeDtypeStruct((nrows, _D), jnp.float32),
        scratch_types=[
            pltpu.VMEM((per_w,), jnp.int32),
            pltpu.VMEM((per_w, _D), jnp.float32),
            pltpu.SemaphoreType.DMA,
        ],
        compiler_params=pltpu.CompilerParams(use_tc_tiling_on_sc=False),
    )
    def gather_kernel(table_hbm, idx_hbm, out_hbm, idx_v, rows_v, sem):
        wid = lax.axis_index("s") * info.num_cores + lax.axis_index("c")
        base = wid * per_w
        pltpu.sync_copy(idx_hbm.at[pl.ds(base, per_w)], idx_v)
        pltpu.async_copy(table_hbm.at[idx_v], rows_v, sem).wait()
        pltpu.sync_copy(rows_v, out_hbm.at[pl.ds(base, per_w)])

    return gather_kernel(table, idx)


def kernel(patient, treatment, confounders, primitive_corpus, W1, b1, W2, b2,
           Wt, bt, Wc, bc, Wa1, ba1, Wa2, ba2, Wo1, bo1, Wo2, bo2, Wo3, bo3,
           Wat1, bat1, Wat2, bat2):
    f32 = jnp.float32
    corpus_pad = jnp.pad(primitive_corpus, ((0, _KPAD - _K), (0, 0)))
    r2 = lambda b: b.reshape(1, -1)

    scores, indices = pl.pallas_call(
        _topk_body,
        grid=(_NB,),
        in_specs=[
            pl.BlockSpec((_B, 48), lambda j: (0, 0)),
            pl.BlockSpec((48, 256), lambda j: (0, 0)),
            pl.BlockSpec((1, 256), lambda j: (0, 0)),
            pl.BlockSpec((256, _D), lambda j: (0, 0)),
            pl.BlockSpec((1, _D), lambda j: (0, 0)),
            pl.BlockSpec((_BK, _D), lambda j: (j, 0)),
        ],
        out_specs=[
            pl.BlockSpec((_B, _TOPK), lambda j: (0, 0)),
            pl.BlockSpec((_B, _TOPK), lambda j: (0, 0)),
        ],
        out_shape=[
            jax.ShapeDtypeStruct((_B, _TOPK), f32),
            jax.ShapeDtypeStruct((_B, _TOPK), jnp.int32),
        ],
        scratch_shapes=[
            pltpu.VMEM((_B, _D), f32),
            pltpu.VMEM((_B, 128), f32),
            pltpu.VMEM((_B, 128), jnp.int32),
        ],
        compiler_params=pltpu.CompilerParams(
            dimension_semantics=("arbitrary",)),
    )(patient, W1, r2(b1), W2, r2(b2), corpus_pad)

    retrieved_flat = _sc_gather(primitive_corpus, indices.reshape(-1))
    retrieved = retrieved_flat.reshape(_B, _TOPK, _D)
    flat = retrieved_flat.reshape(_B, _TOPK * _D)

    outcome, attribution = pl.pallas_call(
        _tail_body,
        in_specs=[pl.BlockSpec(x.shape, lambda: (0,) * x.ndim) for x in (
            flat, treatment, confounders,
            Wt, r2(bt), Wc, r2(bc), Wa1, r2(ba1), Wa2, r2(ba2),
            Wo1, r2(bo1), Wo2, r2(bo2), Wo3, r2(bo3),
            Wat1, r2(bat1), Wat2, r2(bat2))],
        out_specs=[
            pl.BlockSpec((_B, 8), lambda: (0, 0)),
            pl.BlockSpec((_B, 8), lambda: (0, 0)),
        ],
        out_shape=[
            jax.ShapeDtypeStruct((_B, 8), f32),
            jax.ShapeDtypeStruct((_B, 8), f32),
        ],
    )(flat, treatment, confounders,
      Wt, r2(bt), Wc, r2(bc), Wa1, r2(ba1), Wa2, r2(ba2),
      Wo1, r2(bo1), Wo2, r2(bo2), Wo3, r2(bo3),
      Wat1, r2(bat1), Wat2, r2(bat2))

    return (outcome, scores, indices, attribution, retrieved)


# speedup vs baseline: 3.7031x; 1.7861x over previous
import functools

import jax
import jax.numpy as jnp
from jax import lax
from jax.experimental import pallas as pl
from jax.experimental.pallas import tpu as pltpu
from jax.experimental.pallas import tpu_sc as plsc

_B = 1024
_K = 100000
_D = 64
_TOPK = 8
_BK = 2048
_KPAD = 100352
_NB = _KPAD // _BK
_NEG = float("-inf")
_BIGI = 2**30


def _cex(a, b):
    p = a[0] >= b[0]
    hi = (jnp.where(p, a[0], b[0]), jnp.where(p, a[1], b[1]))
    lo = (jnp.where(p, b[0], a[0]), jnp.where(p, b[1], a[1]))
    return hi, lo


def _cex_max(a, b):
    p = a[0] >= b[0]
    return (jnp.where(p, a[0], b[0]), jnp.where(p, a[1], b[1]))


_SORT8 = [(0, 1), (2, 3), (4, 5), (6, 7),
          (0, 2), (1, 3), (4, 6), (5, 7),
          (1, 2), (5, 6),
          (0, 4), (1, 5), (2, 6), (3, 7),
          (2, 4), (3, 5),
          (1, 2), (3, 4), (5, 6)]

_BMERGE8 = [(0, 4), (1, 5), (2, 6), (3, 7),
            (0, 2), (1, 3), (4, 6), (5, 7),
            (0, 1), (2, 3), (4, 5), (6, 7)]


def _sort_net(items, net):
    items = list(items)
    for (i, k) in net:
        items[i], items[k] = _cex(items[i], items[k])
    return items


def _topk_body(patient_ref, w1_ref, b1_ref, w2_ref, b2_ref, corpus_ref,
               scores_ref, idx_ref, pe_ref, ts_ref, ti_ref):
    j = pl.program_id(0)

    @pl.when(j == 0)
    def _init():
        h = jnp.maximum(patient_ref[...] @ w1_ref[...] + b1_ref[...], 0.0)
        pe = h @ w2_ref[...] + b2_ref[...]
        n = jnp.sqrt(jnp.sum(pe * pe, axis=1, keepdims=True))
        pe_ref[...] = pe / jnp.maximum(n, 1e-12)
        ts_ref[...] = jnp.full((_B, 8 * 128), _NEG, jnp.float32)
        ti_ref[...] = jnp.full((_B, 8 * 128), _BIGI, jnp.int32)

    ce = corpus_ref[...]
    n = jnp.sqrt(jnp.sum(ce * ce, axis=1, keepdims=True))
    cen = ce / jnp.maximum(n, 1e-12)
    sim = lax.dot_general(pe_ref[...], cen, (((1,), (1,)), ((), ())),
                          preferred_element_type=jnp.float32)
    col = j * _BK + lax.broadcasted_iota(jnp.int32, (_B, _BK), 1)
    valid = col < _K
    sim = jnp.where(valid, sim, _NEG)
    col = jnp.where(valid, col, _BIGI)

    nch = _BK // 128
    chunks = [(sim[:, c * 128:(c + 1) * 128], col[:, c * 128:(c + 1) * 128])
              for c in range(nch)]
    ha = _sort_net(chunks[0:8], _SORT8)
    hb = _sort_net(chunks[8:16], _SORT8)
    w = [_cex_max(ha[i], hb[7 - i]) for i in range(8)]
    w = _sort_net(w, _BMERGE8)
    r = [(ts_ref[:, t * 128:(t + 1) * 128], ti_ref[:, t * 128:(t + 1) * 128])
         for t in range(8)]
    m = [_cex_max(r[i], w[7 - i]) for i in range(8)]
    m = _sort_net(m, _BMERGE8)

    @pl.when(j < _NB - 1)
    def _store():
        for t in range(8):
            ts_ref[:, t * 128:(t + 1) * 128] = m[t][0]
            ti_ref[:, t * 128:(t + 1) * 128] = m[t][1]

    @pl.when(j == _NB - 1)
    def _emit():
        cand_s = jnp.concatenate([x[0] for x in m], axis=1)
        cand_i = jnp.concatenate([x[1] for x in m], axis=1)
        out_s = []
        out_i = []
        for _ in range(_TOPK):
            mx = jnp.max(cand_s, axis=1, keepdims=True)
            pick = jnp.min(jnp.where(cand_s == mx, cand_i, _BIGI), axis=1,
                           keepdims=True)
            out_s.append(mx)
            out_i.append(pick)
            cand_s = jnp.where(cand_i == pick, _NEG, cand_s)
        scores_ref[...] = jnp.concatenate(out_s, axis=1)
        idx_ref[...] = jnp.concatenate(out_i, axis=1)


def _tail_body(flat_ref, treat_ref, conf_ref,
               wt_ref, bt_ref, wc_ref, bc_ref,
               wa1_ref, ba1_ref, wa2_ref, ba2_ref,
               wo1_ref, bo1_ref, wo2_ref, bo2_ref, wo3_ref, bo3_ref,
               wat1_ref, bat1_ref, wat2_ref, bat2_ref,
               outcome_ref, attr_ref):
    f32 = jnp.float32
    mm = lambda a, b: lax.dot_general(a, b, (((1,), (0,)), ((), ())),
                                      preferred_element_type=f32)
    te = mm(treat_ref[...], wt_ref[...]) + bt_ref[...]
    cenc = mm(conf_ref[...], wc_ref[...]) + bc_ref[...]
    penc = mm(jnp.maximum(mm(flat_ref[...], wa1_ref[...]) + ba1_ref[...], 0.0),
              wa2_ref[...]) + ba2_ref[...]
    h = jnp.maximum(mm(te, wo1_ref[0:256, :]) + mm(cenc, wo1_ref[256:512, :])
                    + mm(penc, wo1_ref[512:768, :]) + bo1_ref[...], 0.0)
    h2 = jnp.maximum(mm(h, wo2_ref[...]) + bo2_ref[...], 0.0)
    outcome = mm(h2, wo3_ref[...]) + bo3_ref[...]
    outcome_ref[...] = outcome
    a = jnp.maximum(mm(penc, wat1_ref[0:256, :])
                    + mm(outcome, wat1_ref[256:264, :]) + bat1_ref[...], 0.0)
    logits = mm(a, wat2_ref[...]) + bat2_ref[...]
    mx = jnp.max(logits, axis=1, keepdims=True)
    e = jnp.exp(logits - mx)
    attr_ref[...] = e / jnp.sum(e, axis=1, keepdims=True)


def _sc_gather(table, idx):
    nrows = idx.shape[0]
    info = plsc.get_sparse_core_info()
    nw = info.num_cores * info.num_subcores
    per_w = nrows // nw
    mesh = plsc.VectorSubcoreMesh(core_axis_name="c", subcore_axis_name="s")

    @functools.partial(
        pl.kernel, mesh=mesh,
        out_type=jax.ShapeDtypeStruct((nrows, _D), jnp.float32),
        scratch_types=[
            pltpu.VMEM((per_w,), jnp.int32),
            pltpu.VMEM((per_w, _D), jnp.float32),
            pltpu.SemaphoreType.DMA,
        ],
        compiler_params=pltpu.CompilerParams(use_tc_tiling_on_sc=False),
    )
    def gather_kernel(table_hbm, idx_hbm, out_hbm, idx_v, rows_v, sem):
        wid = lax.axis_index("s") * info.num_cores + lax.axis_index("c")
        base = wid * per_w
        pltpu.sync_copy(idx_hbm.at[pl.ds(base, per_w)], idx_v)
        pltpu.async_copy(table_hbm.at[idx_v], rows_v, sem).wait()
        pltpu.sync_copy(rows_v, out_hbm.at[pl.ds(base, per_w)])

    return gather_kernel(table, idx)


def kernel(patient, treatment, confounders, primitive_corpus, W1, b1, W2, b2,
           Wt, bt, Wc, bc, Wa1, ba1, Wa2, ba2, Wo1, bo1, Wo2, bo2, Wo3, bo3,
           Wat1, bat1, Wat2, bat2):
    f32 = jnp.float32
    corpus_pad = jnp.pad(primitive_corpus, ((0, _KPAD - _K), (0, 0)))
    r2 = lambda b: b.reshape(1, -1)

    scores, indices = pl.pallas_call(
        _topk_body,
        grid=(_NB,),
        in_specs=[
            pl.BlockSpec((_B, 48), lambda j: (0, 0)),
            pl.BlockSpec((48, 256), lambda j: (0, 0)),
            pl.BlockSpec((1, 256), lambda j: (0, 0)),
            pl.BlockSpec((256, _D), lambda j: (0, 0)),
            pl.BlockSpec((1, _D), lambda j: (0, 0)),
            pl.BlockSpec((_BK, _D), lambda j: (j, 0)),
        ],
        out_specs=[
            pl.BlockSpec((_B, _TOPK), lambda j: (0, 0)),
            pl.BlockSpec((_B, _TOPK), lambda j: (0, 0)),
        ],
        out_shape=[
            jax.ShapeDtypeStruct((_B, _TOPK), f32),
            jax.ShapeDtypeStruct((_B, _TOPK), jnp.int32),
        ],
        scratch_shapes=[
            pltpu.VMEM((_B, _D), f32),
            pltpu.VMEM((_B, 8 * 128), f32),
            pltpu.VMEM((_B, 8 * 128), jnp.int32),
        ],
        compiler_params=pltpu.CompilerParams(
            dimension_semantics=("arbitrary",)),
    )(patient, W1, r2(b1), W2, r2(b2), corpus_pad)

    retrieved_flat = _sc_gather(primitive_corpus, indices.reshape(-1))
    retrieved = retrieved_flat.reshape(_B, _TOPK, _D)
    flat = retrieved_flat.reshape(_B, _TOPK * _D)

    outcome, attribution = pl.pallas_call(
        _tail_body,
        in_specs=[pl.BlockSpec(x.shape, lambda: (0,) * x.ndim) for x in (
            flat, treatment, confounders,
            Wt, r2(bt), Wc, r2(bc), Wa1, r2(ba1), Wa2, r2(ba2),
            Wo1, r2(bo1), Wo2, r2(bo2), Wo3, r2(bo3),
            Wat1, r2(bat1), Wat2, r2(bat2))],
        out_specs=[
            pl.BlockSpec((_B, 8), lambda: (0, 0)),
            pl.BlockSpec((_B, 8), lambda: (0, 0)),
        ],
        out_shape=[
            jax.ShapeDtypeStruct((_B, 8), f32),
            jax.ShapeDtypeStruct((_B, 8), f32),
        ],
    )(flat, treatment, confounders,
      Wt, r2(bt), Wc, r2(bc), Wa1, r2(ba1), Wa2, r2(ba2),
      Wo1, r2(bo1), Wo2, r2(bo2), Wo3, r2(bo3),
      Wat1, r2(bat1), Wat2, r2(bat2))

    return (outcome, scores, indices, attribution, retrieved)


# speedup vs baseline: 4.0229x; 1.0864x over previous
import functools

import jax
import jax.numpy as jnp
from jax import lax
from jax.experimental import pallas as pl
from jax.experimental.pallas import tpu as pltpu
from jax.experimental.pallas import tpu_sc as plsc

_B = 1024
_K = 100000
_D = 64
_TOPK = 8
_BK = 4096
_KPAD = 102400
_NB = _KPAD // _BK
_NEG = float("-inf")
_BIGI = 2**30


def _cex(a, b):
    p = a[0] >= b[0]
    hi = (jnp.where(p, a[0], b[0]), jnp.where(p, a[1], b[1]))
    lo = (jnp.where(p, b[0], a[0]), jnp.where(p, b[1], a[1]))
    return hi, lo


def _cex_max(a, b):
    p = a[0] >= b[0]
    return (jnp.where(p, a[0], b[0]), jnp.where(p, a[1], b[1]))


_SORT8 = [(0, 1), (2, 3), (4, 5), (6, 7),
          (0, 2), (1, 3), (4, 6), (5, 7),
          (1, 2), (5, 6),
          (0, 4), (1, 5), (2, 6), (3, 7),
          (2, 4), (3, 5),
          (1, 2), (3, 4), (5, 6)]

_BMERGE8 = [(0, 4), (1, 5), (2, 6), (3, 7),
            (0, 2), (1, 3), (4, 6), (5, 7),
            (0, 1), (2, 3), (4, 5), (6, 7)]


def _sort_net(items, net):
    items = list(items)
    for (i, k) in net:
        items[i], items[k] = _cex(items[i], items[k])
    return items


def _topk_body(patient_ref, w1_ref, b1_ref, w2_ref, b2_ref, corpus_ref,
               scores_ref, idx_ref, pe_ref, ts_ref, ti_ref):
    j = pl.program_id(0)

    @pl.when(j == 0)
    def _init():
        h = jnp.maximum(patient_ref[...] @ w1_ref[...] + b1_ref[...], 0.0)
        pe = h @ w2_ref[...] + b2_ref[...]
        n = jnp.sqrt(jnp.sum(pe * pe, axis=1, keepdims=True))
        pe_ref[...] = pe / jnp.maximum(n, 1e-12)
        ts_ref[...] = jnp.full((_B, 8 * 128), _NEG, jnp.float32)
        ti_ref[...] = jnp.full((_B, 8 * 128), _BIGI, jnp.int32)

    ce = corpus_ref[...]
    n = jnp.sqrt(jnp.sum(ce * ce, axis=1, keepdims=True))
    cen = ce / jnp.maximum(n, 1e-12)
    sim = lax.dot_general(pe_ref[...], cen, (((1,), (1,)), ((), ())),
                          preferred_element_type=jnp.float32)
    col = j * _BK + lax.broadcasted_iota(jnp.int32, (_B, _BK), 1)
    valid = col < _K
    sim = jnp.where(valid, sim, _NEG)
    col = jnp.where(valid, col, _BIGI)

    nch = _BK // 128
    chunks = [(sim[:, c * 128:(c + 1) * 128], col[:, c * 128:(c + 1) * 128])
              for c in range(nch)]

    def _top8of16(c16):
        ha = _sort_net(c16[0:8], _SORT8)
        hb = _sort_net(c16[8:16], _SORT8)
        w = [_cex_max(ha[i], hb[7 - i]) for i in range(8)]
        return _sort_net(w, _BMERGE8)

    grp = [_top8of16(chunks[g * 16:(g + 1) * 16]) for g in range(nch // 16)]
    w = grp[0]
    for g in grp[1:]:
        w = [_cex_max(w[i], g[7 - i]) for i in range(8)]
        w = _sort_net(w, _BMERGE8)
    r = [(ts_ref[:, t * 128:(t + 1) * 128], ti_ref[:, t * 128:(t + 1) * 128])
         for t in range(8)]
    m = [_cex_max(r[i], w[7 - i]) for i in range(8)]
    m = _sort_net(m, _BMERGE8)

    @pl.when(j < _NB - 1)
    def _store():
        for t in range(8):
            ts_ref[:, t * 128:(t + 1) * 128] = m[t][0]
            ti_ref[:, t * 128:(t + 1) * 128] = m[t][1]

    @pl.when(j == _NB - 1)
    def _emit():
        cand_s = jnp.concatenate([x[0] for x in m], axis=1)
        cand_i = jnp.concatenate([x[1] for x in m], axis=1)
        out_s = []
        out_i = []
        for _ in range(_TOPK):
            mx = jnp.max(cand_s, axis=1, keepdims=True)
            pick = jnp.min(jnp.where(cand_s == mx, cand_i, _BIGI), axis=1,
                           keepdims=True)
            out_s.append(mx)
            out_i.append(pick)
            cand_s = jnp.where(cand_i == pick, _NEG, cand_s)
        scores_ref[...] = jnp.concatenate(out_s, axis=1)
        idx_ref[...] = jnp.concatenate(out_i, axis=1)


def _tail_body(flat_ref, treat_ref, conf_ref,
               wt_ref, bt_ref, wc_ref, bc_ref,
               wa1_ref, ba1_ref, wa2_ref, ba2_ref,
               wo1_ref, bo1_ref, wo2_ref, bo2_ref, wo3_ref, bo3_ref,
               wat1_ref, bat1_ref, wat2_ref, bat2_ref,
               outcome_ref, attr_ref):
    f32 = jnp.float32
    mm = lambda a, b: lax.dot_general(a, b, (((1,), (0,)), ((), ())),
                                      preferred_element_type=f32)
    te = mm(treat_ref[...], wt_ref[...]) + bt_ref[...]
    cenc = mm(conf_ref[...], wc_ref[...]) + bc_ref[...]
    penc = mm(jnp.maximum(mm(flat_ref[...], wa1_ref[...]) + ba1_ref[...], 0.0),
              wa2_ref[...]) + ba2_ref[...]
    h = jnp.maximum(mm(te, wo1_ref[0:256, :]) + mm(cenc, wo1_ref[256:512, :])
                    + mm(penc, wo1_ref[512:768, :]) + bo1_ref[...], 0.0)
    h2 = jnp.maximum(mm(h, wo2_ref[...]) + bo2_ref[...], 0.0)
    outcome = mm(h2, wo3_ref[...]) + bo3_ref[...]
    outcome_ref[...] = outcome
    a = jnp.maximum(mm(penc, wat1_ref[0:256, :])
                    + mm(outcome, wat1_ref[256:264, :]) + bat1_ref[...], 0.0)
    logits = mm(a, wat2_ref[...]) + bat2_ref[...]
    mx = jnp.max(logits, axis=1, keepdims=True)
    e = jnp.exp(logits - mx)
    attr_ref[...] = e / jnp.sum(e, axis=1, keepdims=True)


def _sc_gather(table, idx):
    nrows = idx.shape[0]
    info = plsc.get_sparse_core_info()
    nw = info.num_cores * info.num_subcores
    per_w = nrows // nw
    mesh = plsc.VectorSubcoreMesh(core_axis_name="c", subcore_axis_name="s")

    @functools.partial(
        pl.kernel, mesh=mesh,
        out_type=jax.ShapeDtypeStruct((nrows, _D), jnp.float32),
        scratch_types=[
            pltpu.VMEM((per_w,), jnp.int32),
            pltpu.VMEM((per_w, _D), jnp.float32),
            pltpu.SemaphoreType.DMA,
        ],
        compiler_params=pltpu.CompilerParams(use_tc_tiling_on_sc=False),
    )
    def gather_kernel(table_hbm, idx_hbm, out_hbm, idx_v, rows_v, sem):
        wid = lax.axis_index("s") * info.num_cores + lax.axis_index("c")
        base = wid * per_w
        pltpu.sync_copy(idx_hbm.at[pl.ds(base, per_w)], idx_v)
        pltpu.async_copy(table_hbm.at[idx_v], rows_v, sem).wait()
        pltpu.sync_copy(rows_v, out_hbm.at[pl.ds(base, per_w)])

    return gather_kernel(table, idx)


def kernel(patient, treatment, confounders, primitive_corpus, W1, b1, W2, b2,
           Wt, bt, Wc, bc, Wa1, ba1, Wa2, ba2, Wo1, bo1, Wo2, bo2, Wo3, bo3,
           Wat1, bat1, Wat2, bat2):
    f32 = jnp.float32
    r2 = lambda b: b.reshape(1, -1)

    scores, indices = pl.pallas_call(
        _topk_body,
        grid=(_NB,),
        in_specs=[
            pl.BlockSpec((_B, 48), lambda j: (0, 0)),
            pl.BlockSpec((48, 256), lambda j: (0, 0)),
            pl.BlockSpec((1, 256), lambda j: (0, 0)),
            pl.BlockSpec((256, _D), lambda j: (0, 0)),
            pl.BlockSpec((1, _D), lambda j: (0, 0)),
            pl.BlockSpec((_BK, _D), lambda j: (j, 0)),
        ],
        out_specs=[
            pl.BlockSpec((_B, _TOPK), lambda j: (0, 0)),
            pl.BlockSpec((_B, _TOPK), lambda j: (0, 0)),
        ],
        out_shape=[
            jax.ShapeDtypeStruct((_B, _TOPK), f32),
            jax.ShapeDtypeStruct((_B, _TOPK), jnp.int32),
        ],
        scratch_shapes=[
            pltpu.VMEM((_B, _D), f32),
            pltpu.VMEM((_B, 8 * 128), f32),
            pltpu.VMEM((_B, 8 * 128), jnp.int32),
        ],
        compiler_params=pltpu.CompilerParams(
            dimension_semantics=("arbitrary",)),
    )(patient, W1, r2(b1), W2, r2(b2), primitive_corpus)

    retrieved_flat = _sc_gather(primitive_corpus, indices.reshape(-1))
    retrieved = retrieved_flat.reshape(_B, _TOPK, _D)
    flat = retrieved_flat.reshape(_B, _TOPK * _D)

    outcome, attribution = pl.pallas_call(
        _tail_body,
        in_specs=[pl.BlockSpec(x.shape, lambda: (0,) * x.ndim) for x in (
            flat, treatment, confounders,
            Wt, r2(bt), Wc, r2(bc), Wa1, r2(ba1), Wa2, r2(ba2),
            Wo1, r2(bo1), Wo2, r2(bo2), Wo3, r2(bo3),
            Wat1, r2(bat1), Wat2, r2(bat2))],
        out_specs=[
            pl.BlockSpec((_B, 8), lambda: (0, 0)),
            pl.BlockSpec((_B, 8), lambda: (0, 0)),
        ],
        out_shape=[
            jax.ShapeDtypeStruct((_B, 8), f32),
            jax.ShapeDtypeStruct((_B, 8), f32),
        ],
    )(flat, treatment, confounders,
      Wt, r2(bt), Wc, r2(bc), Wa1, r2(ba1), Wa2, r2(ba2),
      Wo1, r2(bo1), Wo2, r2(bo2), Wo3, r2(bo3),
      Wat1, r2(bat1), Wat2, r2(bat2))

    return (outcome, scores, indices, attribution, retrieved)
